# baseline (speedup 1.0000x reference)
import functools

import jax
import jax.numpy as jnp
from jax.experimental import pallas as pl
from jax.experimental.pallas import tpu as pltpu


_TAPS = (
    (0, 3, 1, 1),
    (1, 2, 1, 0),
    (2, 3, 1, 0),
    (3, 1, 0, 1),
    (4, 0, 0, 0),
    (5, 1, 0, 0),
    (6, 3, 0, 1),
    (7, 2, 0, 0),
    (8, 3, 0, 0),
)


def _phase_matmul_kernel(p_ref, w_ref, b_ref, o_ref, *, wo_sz):
    n_lanes = p_ref.shape[-1]
    lane = jax.lax.broadcasted_iota(jnp.int32, (1, n_lanes), 1)
    col_ok = (lane % wo_sz) != 0

    acc = None
    for t, phase, sh, sw in _TAPS:
        v = p_ref[0, phase].astype(jnp.bfloat16)
        shift = sh * wo_sz + sw
        if shift:
            pad = jnp.zeros((v.shape[0], shift), v.dtype)
            v = jnp.concatenate([pad, v[:, :-shift]], axis=1)
        if sw:
            v = jnp.where(col_ok, v, jnp.zeros((), v.dtype))
        d = jnp.dot(w_ref[t], v, preferred_element_type=jnp.float32)
        acc = d if acc is None else acc + d
    o_ref[0] = acc + b_ref[...]


def _resident_spec(shape):
    index_map = lambda *_: (0,) * len(shape)
    try:
        return pl.BlockSpec(shape, index_map, pipeline_mode=pl.Buffered(1))
    except (TypeError, AttributeError):
        return pl.BlockSpec(shape, index_map)


@jax.jit
def _downsample_conv(x, w_mat, bias_row):
    B, C, H, W = x.shape
    Co = w_mat.shape[1]
    Ho, Wo = H // 2, W // 2
    P = Ho * Wo

    xr = x.reshape(B, C, Ho, 2, Wo, 2)
    phases = jnp.transpose(xr, (0, 3, 5, 1, 2, 4)).reshape(B, 4, C, P)

    w_taps = jnp.transpose(w_mat.reshape(9, C, Co), (0, 2, 1))
    w_taps = w_taps.astype(jnp.bfloat16)
    bias_col = jnp.transpose(bias_row, (1, 0)).astype(jnp.float32)

    flops = 2 * B * P * 9 * C * Co
    bytes_accessed = phases.size * 2 + w_taps.size * 2 + B * Co * P * 4
    out_flat = pl.pallas_call(
        functools.partial(_phase_matmul_kernel, wo_sz=Wo),
        out_shape=jax.ShapeDtypeStruct((B, Co, P), jnp.float32),
        grid=(B,),
        in_specs=[
            pl.BlockSpec((1, 4, C, P), lambda b: (b, 0, 0, 0)),
            _resident_spec((9, Co, C)),
            _resident_spec((Co, 1)),
        ],
        out_specs=pl.BlockSpec((1, Co, P), lambda b: (b, 0, 0)),
        compiler_params=pltpu.CompilerParams(
            dimension_semantics=("parallel",),
            vmem_limit_bytes=48 << 20,
        ),
        cost_estimate=pl.CostEstimate(
            flops=flops, transcendentals=0, bytes_accessed=bytes_accessed),
    )(phases, w_taps, bias_col)

    return out_flat.reshape(B, Co, Ho, Wo)


def kernel(x, w_mat, bias_row):
    return _downsample_conv(x, w_mat, bias_row)


# speedup vs baseline: 1.5161x; 1.5161x over previous
import functools

import jax
import jax.numpy as jnp
from jax.experimental import pallas as pl
from jax.experimental.pallas import tpu as pltpu


_TAPS = (
    (0, 3, 1, 1),
    (1, 2, 1, 0),
    (2, 3, 1, 0),
    (3, 1, 0, 1),
    (4, 0, 0, 0),
    (5, 1, 0, 0),
    (6, 3, 0, 1),
    (7, 2, 0, 0),
    (8, 3, 0, 0),
)


def _conv_kernel(x_ref, w_ref, b_ref, o_ref, *, wo_sz):
    w_sz = 2 * wo_sz
    strip = 2 * w_sz
    n_strips = x_ref.shape[-1] // strip

    y = x_ref[0].astype(jnp.bfloat16)
    z = pltpu.bitcast(y, jnp.int32)
    rows = z.shape[0]

    j = jax.lax.broadcasted_iota(jnp.int32, (rows, strip), 1)
    r = j % wo_sz
    q = j // wo_sz
    idx = 2 * wo_sz * (q // 2) + 2 * r + (q % 2)
    deint = [
        jnp.take_along_axis(z[:, k * strip:(k + 1) * strip], idx, axis=1)
        for k in range(n_strips)
    ]
    phases = tuple(
        pltpu.bitcast(
            jnp.concatenate(
                [d[:, q * wo_sz:(q + 1) * wo_sz] for d in deint], axis=1),
            jnp.bfloat16)
        for q in range(4))

    n_lanes = n_strips * wo_sz
    lane = jax.lax.broadcasted_iota(jnp.int32, (1, n_lanes), 1)
    col_ok = (lane % wo_sz) != 0

    acc = None
    for t, phase, sh, sw in _TAPS:
        v = phases[phase]
        shift = sh * wo_sz + sw
        if shift:
            pad = jnp.zeros((v.shape[0], shift), v.dtype)
            v = jnp.concatenate([pad, v[:, :-shift]], axis=1)
        if sw:
            v = jnp.where(col_ok, v, jnp.zeros((), v.dtype))
        d = jnp.dot(w_ref[t], v, preferred_element_type=jnp.float32)
        acc = d if acc is None else acc + d
    o_ref[0] = acc + b_ref[...]


def _resident_spec(shape):
    index_map = lambda *_: (0,) * len(shape)
    try:
        return pl.BlockSpec(shape, index_map, pipeline_mode=pl.Buffered(1))
    except (TypeError, AttributeError):
        return pl.BlockSpec(shape, index_map)


@jax.jit
def _downsample_conv(x, w_mat, bias_row):
    B, C, H, W = x.shape
    Co = w_mat.shape[1]
    Ho, Wo = H // 2, W // 2
    P = Ho * Wo

    x_flat = x.reshape(B, C, H * W)

    w_taps = jnp.transpose(w_mat.reshape(9, C, Co), (0, 2, 1)).astype(jnp.bfloat16)
    bias_col = jnp.transpose(bias_row, (1, 0)).astype(jnp.float32)

    flops = 2 * B * P * 9 * C * Co
    bytes_accessed = x_flat.size * 4 + w_taps.size * 2 + B * Co * P * 4
    out_flat = pl.pallas_call(
        functools.partial(_conv_kernel, wo_sz=Wo),
        out_shape=jax.ShapeDtypeStruct((B, Co, P), jnp.float32),
        grid=(B,),
        in_specs=[
            pl.BlockSpec((1, C, H * W), lambda b: (b, 0, 0)),
            _resident_spec((9, Co, C)),
            _resident_spec((Co, 1)),
        ],
        out_specs=pl.BlockSpec((1, Co, P), lambda b: (b, 0, 0)),
        compiler_params=pltpu.CompilerParams(
            dimension_semantics=("parallel",),
            vmem_limit_bytes=48 << 20,
        ),
        cost_estimate=pl.CostEstimate(
            flops=flops, transcendentals=0, bytes_accessed=bytes_accessed),
    )(x_flat, w_taps, bias_col)

    return out_flat.reshape(B, Co, Ho, Wo)


def kernel(x, w_mat, bias_row):
    return _downsample_conv(x, w_mat, bias_row)
